# baseline (device time: 84246 ns/iter reference)
import jax
import jax.numpy as jnp
from jax import lax
from jax.experimental import pallas as pl
from jax.experimental.pallas import tpu as pltpu

N_DEV = 16
N_TOK = 2048
D_MODEL = 512
D_HID = 1024
N_EXP = 128
EXP_PER_DEV = N_EXP // N_DEV
CAP = 12
ROWS = EXP_PER_DEV * CAP
TOK_PER_DEV = N_TOK // N_DEV


def _compute_and_allgather(xg, w):

    def body(xg_ref, w_ref, out_ref, send_sems, recv_sems):
        my = lax.axis_index("i")
        left = lax.rem(my - 1 + N_DEV, N_DEV)
        right = lax.rem(my + 1, N_DEV)

        blocks = []
        for e in range(EXP_PER_DEV):
            blocks.append(
                jnp.dot(
                    xg_ref[e * CAP:(e + 1) * CAP, :],
                    w_ref[e],
                    preferred_element_type=jnp.float32,
                )
            )
        block = jnp.concatenate(blocks, axis=0).astype(out_ref.dtype)
        out_ref[pl.ds(my, 1)] = block[None]

        barrier_sem = pltpu.get_barrier_semaphore()
        for nbr in (left, right):
            pl.semaphore_signal(
                barrier_sem, inc=1,
                device_id=(nbr,), device_id_type=pl.DeviceIdType.MESH,
            )
        pl.semaphore_wait(barrier_sem, 2)

        for h in range(N_DEV - 1):
            slot = lax.rem(my - h + N_DEV, N_DEV)
            rdma = pltpu.make_async_remote_copy(
                src_ref=out_ref.at[pl.ds(slot, 1)],
                dst_ref=out_ref.at[pl.ds(slot, 1)],
                send_sem=send_sems.at[h],
                recv_sem=recv_sems.at[h],
                device_id=(right,),
                device_id_type=pl.DeviceIdType.MESH,
            )
            rdma.start()
            rdma.wait()

    return pl.pallas_call(
        body,
        out_shape=jax.ShapeDtypeStruct((N_DEV, ROWS, D_HID), jnp.bfloat16),
        in_specs=[
            pl.BlockSpec(memory_space=pltpu.VMEM),
            pl.BlockSpec(memory_space=pltpu.VMEM),
        ],
        out_specs=pl.BlockSpec(memory_space=pltpu.VMEM),
        scratch_shapes=[
            pltpu.SemaphoreType.DMA((N_DEV - 1,)),
            pltpu.SemaphoreType.DMA((N_DEV - 1,)),
        ],
        compiler_params=pltpu.CompilerParams(collective_id=0),
    )(xg, w)


def kernel(x, router_W, route_idx, expert_W):
    del router_W

    my = lax.axis_index("i")
    r = route_idx[:, 0].astype(jnp.int32)

    onehot = r[:, None] == jnp.arange(N_EXP, dtype=jnp.int32)[None, :]
    cum = jnp.cumsum(onehot.astype(jnp.int32), axis=0) - 1
    rank = jnp.sum(jnp.where(onehot, cum, 0), axis=1)

    token_table = jnp.full((N_EXP, CAP), N_TOK, dtype=jnp.int32)
    token_table = token_table.at[r, rank].set(
        jnp.arange(N_TOK, dtype=jnp.int32), mode="drop"
    )

    my_tok = lax.dynamic_slice(
        token_table, (my * EXP_PER_DEV, 0), (EXP_PER_DEV, CAP)
    ).reshape(ROWS)
    valid = my_tok < N_TOK
    xg = x[jnp.where(valid, my_tok, 0)] * valid[:, None].astype(x.dtype)
    xg = xg.astype(jnp.bfloat16)

    table = _compute_and_allgather(xg, expert_W.astype(jnp.bfloat16))
    table = table.reshape(N_DEV * ROWS, D_HID)

    t0 = my * TOK_PER_DEV
    e_mine = lax.dynamic_slice(r, (t0,), (TOK_PER_DEV,))
    rank_mine = lax.dynamic_slice(rank, (t0,), (TOK_PER_DEV,))
    active = rank_mine < CAP
    flat = jnp.where(active, e_mine * CAP + rank_mine, 0)
    out = jnp.where(active[:, None], table[flat].astype(jnp.float32), 0.0)
    return out


# device time: 69015 ns/iter; 1.2207x vs baseline; 1.2207x over previous
import jax
import jax.numpy as jnp
from jax import lax
from jax.experimental import pallas as pl
from jax.experimental.pallas import tpu as pltpu

N_DEV = 16
N_TOK = 2048
D_MODEL = 512
D_HID = 1024
N_EXP = 128
EXP_PER_DEV = N_EXP // N_DEV
CAP = 12
ROWS = EXP_PER_DEV * CAP
TOK_PER_DEV = N_TOK // N_DEV


def _compute_and_allgather(xg, w):

    def body(xg_ref, w_ref, out_ref, sr_sems, rr_sems, sl_sems, rl_sems):
        my = lax.axis_index("i")
        left = lax.rem(my - 1 + N_DEV, N_DEV)
        right = lax.rem(my + 1, N_DEV)

        blocks = []
        for e in range(EXP_PER_DEV):
            blocks.append(
                jnp.dot(
                    xg_ref[e * CAP:(e + 1) * CAP, :],
                    w_ref[e],
                    preferred_element_type=jnp.float32,
                )
            )
        block = jnp.concatenate(blocks, axis=0).astype(out_ref.dtype)
        out_ref[pl.ds(my, 1)] = block[None]

        barrier_sem = pltpu.get_barrier_semaphore()
        for nbr in (left, right):
            pl.semaphore_signal(
                barrier_sem, inc=1,
                device_id=(nbr,), device_id_type=pl.DeviceIdType.MESH,
            )
        pl.semaphore_wait(barrier_sem, 2)

        n_right = N_DEV // 2
        n_left = N_DEV - 1 - n_right
        for h in range(n_right):
            slot_r = lax.rem(my - h + N_DEV, N_DEV)
            rdma_r = pltpu.make_async_remote_copy(
                src_ref=out_ref.at[pl.ds(slot_r, 1)],
                dst_ref=out_ref.at[pl.ds(slot_r, 1)],
                send_sem=sr_sems.at[h],
                recv_sem=rr_sems.at[h],
                device_id=(right,),
                device_id_type=pl.DeviceIdType.MESH,
            )
            rdma_r.start()
            rdma_l = None
            if h < n_left:
                slot_l = lax.rem(my + h, N_DEV)
                rdma_l = pltpu.make_async_remote_copy(
                    src_ref=out_ref.at[pl.ds(slot_l, 1)],
                    dst_ref=out_ref.at[pl.ds(slot_l, 1)],
                    send_sem=sl_sems.at[h],
                    recv_sem=rl_sems.at[h],
                    device_id=(left,),
                    device_id_type=pl.DeviceIdType.MESH,
                )
                rdma_l.start()
            rdma_r.wait()
            if rdma_l is not None:
                rdma_l.wait()

    return pl.pallas_call(
        body,
        out_shape=jax.ShapeDtypeStruct((N_DEV, ROWS, D_HID), jnp.bfloat16),
        in_specs=[
            pl.BlockSpec(memory_space=pltpu.VMEM),
            pl.BlockSpec(memory_space=pltpu.VMEM),
        ],
        out_specs=pl.BlockSpec(memory_space=pltpu.VMEM),
        scratch_shapes=[
            pltpu.SemaphoreType.DMA((N_DEV // 2,)),
            pltpu.SemaphoreType.DMA((N_DEV // 2,)),
            pltpu.SemaphoreType.DMA((N_DEV - 1 - N_DEV // 2,)),
            pltpu.SemaphoreType.DMA((N_DEV - 1 - N_DEV // 2,)),
        ],
        compiler_params=pltpu.CompilerParams(collective_id=0),
    )(xg, w)


def kernel(x, router_W, route_idx, expert_W):
    del router_W

    my = lax.axis_index("i")
    r = route_idx[:, 0].astype(jnp.int32)

    onehot = r[:, None] == jnp.arange(N_EXP, dtype=jnp.int32)[None, :]
    cum = jnp.cumsum(onehot.astype(jnp.int32), axis=0) - 1
    rank = jnp.sum(jnp.where(onehot, cum, 0), axis=1)

    token_table = jnp.full((N_EXP, CAP), N_TOK, dtype=jnp.int32)
    token_table = token_table.at[r, rank].set(
        jnp.arange(N_TOK, dtype=jnp.int32), mode="drop"
    )

    my_tok = lax.dynamic_slice(
        token_table, (my * EXP_PER_DEV, 0), (EXP_PER_DEV, CAP)
    ).reshape(ROWS)
    valid = my_tok < N_TOK
    xg = x[jnp.where(valid, my_tok, 0)] * valid[:, None].astype(x.dtype)
    xg = xg.astype(jnp.bfloat16)

    table = _compute_and_allgather(xg, expert_W.astype(jnp.bfloat16))
    table = table.reshape(N_DEV * ROWS, D_HID)

    t0 = my * TOK_PER_DEV
    e_mine = lax.dynamic_slice(r, (t0,), (TOK_PER_DEV,))
    rank_mine = lax.dynamic_slice(rank, (t0,), (TOK_PER_DEV,))
    active = rank_mine < CAP
    flat = jnp.where(active, e_mine * CAP + rank_mine, 0)
    out = jnp.where(active[:, None], table[flat].astype(jnp.float32), 0.0)
    return out


# device time: 68909 ns/iter; 1.2226x vs baseline; 1.0015x over previous
import functools

import jax
import jax.numpy as jnp
from jax import lax
from jax.experimental import pallas as pl
from jax.experimental.pallas import tpu as pltpu

N_DEV = 16
N_TOK = 2048
D_MODEL = 512
D_HID = 1024
N_EXP = 128
EXP_PER_DEV = N_EXP // N_DEV
CAP = 12
ROWS = EXP_PER_DEV * CAP
TOK_PER_DEV = N_TOK // N_DEV


def _compute_and_allgather(xg, w):

    def body(xg_ref, w_ref, out_ref, send_sems, recv_sems):
        my = lax.axis_index("i")

        blocks = []
        for e in range(EXP_PER_DEV):
            blocks.append(
                jnp.dot(
                    xg_ref[e * CAP:(e + 1) * CAP, :],
                    w_ref[e],
                    preferred_element_type=jnp.float32,
                )
            )
        block = jnp.concatenate(blocks, axis=0).astype(out_ref.dtype)
        out_ref[pl.ds(my, 1)] = block[None]

        barrier_sem = pltpu.get_barrier_semaphore()
        for k in range(1, N_DEV):
            pl.semaphore_signal(
                barrier_sem, inc=1,
                device_id=(lax.rem(my + k, N_DEV),),
                device_id_type=pl.DeviceIdType.MESH,
            )
        pl.semaphore_wait(barrier_sem, N_DEV - 1)

        rdmas = []
        for k in range(1, N_DEV):
            rd = pltpu.make_async_remote_copy(
                src_ref=out_ref.at[pl.ds(my, 1)],
                dst_ref=out_ref.at[pl.ds(my, 1)],
                send_sem=send_sems.at[k - 1],
                recv_sem=recv_sems.at[k - 1],
                device_id=(lax.rem(my + k, N_DEV),),
                device_id_type=pl.DeviceIdType.MESH,
            )
            rd.start()
            rdmas.append(rd)
        for rd in rdmas:
            rd.wait()

        @functools.partial(
            pl.run_scoped, second_barrier=pltpu.SemaphoreType.REGULAR
        )
        def _(second_barrier):
            for k in range(1, N_DEV):
                pl.semaphore_signal(
                    second_barrier, inc=1,
                    device_id=(lax.rem(my + k, N_DEV),),
                    device_id_type=pl.DeviceIdType.MESH,
                )
            pl.semaphore_wait(second_barrier, N_DEV - 1)

    return pl.pallas_call(
        body,
        out_shape=jax.ShapeDtypeStruct((N_DEV, ROWS, D_HID), jnp.bfloat16),
        in_specs=[
            pl.BlockSpec(memory_space=pltpu.VMEM),
            pl.BlockSpec(memory_space=pltpu.VMEM),
        ],
        out_specs=pl.BlockSpec(memory_space=pltpu.VMEM),
        scratch_shapes=[
            pltpu.SemaphoreType.DMA((N_DEV - 1,)),
            pltpu.SemaphoreType.DMA((N_DEV - 1,)),
        ],
        compiler_params=pltpu.CompilerParams(collective_id=0),
    )(xg, w)


def kernel(x, router_W, route_idx, expert_W):
    del router_W

    my = lax.axis_index("i")
    r = route_idx[:, 0].astype(jnp.int32)

    onehot = r[:, None] == jnp.arange(N_EXP, dtype=jnp.int32)[None, :]
    cum = jnp.cumsum(onehot.astype(jnp.int32), axis=0) - 1
    rank = jnp.sum(jnp.where(onehot, cum, 0), axis=1)

    token_table = jnp.full((N_EXP, CAP), N_TOK, dtype=jnp.int32)
    token_table = token_table.at[r, rank].set(
        jnp.arange(N_TOK, dtype=jnp.int32), mode="drop"
    )

    my_tok = lax.dynamic_slice(
        token_table, (my * EXP_PER_DEV, 0), (EXP_PER_DEV, CAP)
    ).reshape(ROWS)
    valid = my_tok < N_TOK
    xg = x[jnp.where(valid, my_tok, 0)] * valid[:, None].astype(x.dtype)
    xg = xg.astype(jnp.bfloat16)

    table = _compute_and_allgather(xg, expert_W.astype(jnp.bfloat16))
    table = table.reshape(N_DEV * ROWS, D_HID)

    t0 = my * TOK_PER_DEV
    e_mine = lax.dynamic_slice(r, (t0,), (TOK_PER_DEV,))
    rank_mine = lax.dynamic_slice(rank, (t0,), (TOK_PER_DEV,))
    active = rank_mine < CAP
    flat = jnp.where(active, e_mine * CAP + rank_mine, 0)
    out = jnp.where(active[:, None], table[flat].astype(jnp.float32), 0.0)
    return out


# device time: 45767 ns/iter; 1.8408x vs baseline; 1.5056x over previous
import functools

import jax
import jax.numpy as jnp
from jax import lax
from jax.experimental import pallas as pl
from jax.experimental.pallas import tpu as pltpu

N_DEV = 16
N_TOK = 2048
D_MODEL = 512
D_HID = 1024
N_EXP = 128
EXP_PER_DEV = N_EXP // N_DEV
CAP = 12
ROWS = EXP_PER_DEV * CAP
TOK_PER_DEV = N_TOK // N_DEV


def _compute_and_scatter(my_tok, expect, xg, w):

    def body(tok_ref, exp_ref, xg_ref, w_ref, out_ref, rows2d_ref,
             rows_ref, send_sems, recv_sems):
        my = lax.axis_index("i")

        for e in range(EXP_PER_DEV):
            rows2d_ref[e * CAP:(e + 1) * CAP, :] = jnp.dot(
                xg_ref[e * CAP:(e + 1) * CAP, :],
                w_ref[e],
                preferred_element_type=jnp.float32,
            ).astype(rows2d_ref.dtype)
        for row in range(ROWS):
            rows_ref[row, :, :] = rows2d_ref[pl.ds(row, 1), :]

        out_ref[:, :, :] = jnp.zeros((TOK_PER_DEV, 1, D_HID), out_ref.dtype)

        for row in range(ROWS):
            t = tok_ref[row]
            valid = t < N_TOK
            ts = jnp.where(valid, t, 0)
            dest = ts // TOK_PER_DEV
            j = lax.rem(ts, TOK_PER_DEV)

            @pl.when(valid & (dest == my))
            def _(row=row, j=j):
                out_ref[pl.ds(j, 1)] = rows_ref[pl.ds(row, 1)]

        barrier_sem = pltpu.get_barrier_semaphore()
        for k in range(1, N_DEV):
            pl.semaphore_signal(
                barrier_sem, inc=1,
                device_id=(lax.rem(my + k, N_DEV),),
                device_id_type=pl.DeviceIdType.MESH,
            )
        pl.semaphore_wait(barrier_sem, N_DEV - 1)

        for row in range(ROWS):
            t = tok_ref[row]
            valid = t < N_TOK
            ts = jnp.where(valid, t, 0)
            dest = ts // TOK_PER_DEV
            j = lax.rem(ts, TOK_PER_DEV)

            @pl.when(valid & (dest != my))
            def _(row=row, j=j, dest=dest):
                pltpu.make_async_remote_copy(
                    src_ref=rows_ref.at[pl.ds(row, 1)],
                    dst_ref=out_ref.at[pl.ds(j, 1)],
                    send_sem=send_sems.at[row],
                    recv_sem=recv_sems.at[j],
                    device_id=(dest,),
                    device_id_type=pl.DeviceIdType.MESH,
                ).start()

        for j in range(TOK_PER_DEV):
            @pl.when(exp_ref[j] != 0)
            def _(j=j):
                pltpu.make_async_remote_copy(
                    src_ref=rows_ref.at[pl.ds(0, 1)],
                    dst_ref=out_ref.at[pl.ds(j, 1)],
                    send_sem=send_sems.at[0],
                    recv_sem=recv_sems.at[j],
                    device_id=(my,),
                    device_id_type=pl.DeviceIdType.MESH,
                ).wait_recv()

        for row in range(ROWS):
            t = tok_ref[row]
            valid = t < N_TOK
            ts = jnp.where(valid, t, 0)
            dest = ts // TOK_PER_DEV

            @pl.when(valid & (dest != my))
            def _(row=row):
                pltpu.make_async_remote_copy(
                    src_ref=rows_ref.at[pl.ds(row, 1)],
                    dst_ref=out_ref.at[pl.ds(0, 1)],
                    send_sem=send_sems.at[row],
                    recv_sem=recv_sems.at[0],
                    device_id=(my,),
                    device_id_type=pl.DeviceIdType.MESH,
                ).wait_send()

        @functools.partial(
            pl.run_scoped, second_barrier=pltpu.SemaphoreType.REGULAR
        )
        def _(second_barrier):
            for k in range(1, N_DEV):
                pl.semaphore_signal(
                    second_barrier, inc=1,
                    device_id=(lax.rem(my + k, N_DEV),),
                    device_id_type=pl.DeviceIdType.MESH,
                )
            pl.semaphore_wait(second_barrier, N_DEV - 1)

    return pl.pallas_call(
        body,
        out_shape=jax.ShapeDtypeStruct((TOK_PER_DEV, 1, D_HID), jnp.bfloat16),
        in_specs=[
            pl.BlockSpec(memory_space=pltpu.SMEM),
            pl.BlockSpec(memory_space=pltpu.SMEM),
            pl.BlockSpec(memory_space=pltpu.VMEM),
            pl.BlockSpec(memory_space=pltpu.VMEM),
        ],
        out_specs=pl.BlockSpec(memory_space=pltpu.VMEM),
        scratch_shapes=[
            pltpu.VMEM((ROWS, D_HID), jnp.bfloat16),
            pltpu.VMEM((ROWS, 1, D_HID), jnp.bfloat16),
            pltpu.SemaphoreType.DMA((ROWS,)),
            pltpu.SemaphoreType.DMA((TOK_PER_DEV,)),
        ],
        compiler_params=pltpu.CompilerParams(collective_id=0),
    )(my_tok, expect, xg, w)


def kernel(x, router_W, route_idx, expert_W):
    del router_W

    my = lax.axis_index("i")
    r = route_idx[:, 0].astype(jnp.int32)

    onehot = r[:, None] == jnp.arange(N_EXP, dtype=jnp.int32)[None, :]
    cum = jnp.cumsum(onehot.astype(jnp.int32), axis=0) - 1
    rank = jnp.sum(jnp.where(onehot, cum, 0), axis=1)

    token_table = jnp.full((N_EXP, CAP), N_TOK, dtype=jnp.int32)
    token_table = token_table.at[r, rank].set(
        jnp.arange(N_TOK, dtype=jnp.int32), mode="drop"
    )

    my_tok = lax.dynamic_slice(
        token_table, (my * EXP_PER_DEV, 0), (EXP_PER_DEV, CAP)
    ).reshape(ROWS)
    valid = my_tok < N_TOK
    xg = x[jnp.where(valid, my_tok, 0)] * valid[:, None].astype(x.dtype)
    xg = xg.astype(jnp.bfloat16)

    t0 = my * TOK_PER_DEV
    e_mine = lax.dynamic_slice(r, (t0,), (TOK_PER_DEV,))
    rank_mine = lax.dynamic_slice(rank, (t0,), (TOK_PER_DEV,))
    active = rank_mine < CAP
    remote = active & (e_mine // EXP_PER_DEV != my)
    expect = remote.astype(jnp.int32)

    out = _compute_and_scatter(
        my_tok, expect, xg, expert_W.astype(jnp.bfloat16)
    )
    return out.reshape(TOK_PER_DEV, D_HID).astype(jnp.float32)


# device time: 33695 ns/iter; 2.5003x vs baseline; 1.3583x over previous
import functools

import jax
import jax.numpy as jnp
from jax import lax
from jax.experimental import pallas as pl
from jax.experimental.pallas import tpu as pltpu

N_DEV = 16
N_TOK = 2048
D_MODEL = 512
D_HID = 1024
N_EXP = 128
EXP_PER_DEV = N_EXP // N_DEV
CAP = 12
ROWS = EXP_PER_DEV * CAP
TOK_PER_DEV = N_TOK // N_DEV


def _compute_and_scatter(my_tok, expect, xg, w):

    def body(tok_ref, exp_ref, xg_ref, w_ref, out_ref, rows2d_ref,
             rows_ref, send_sems, recv_sems):
        my = lax.axis_index("i")

        for e in range(EXP_PER_DEV):
            rows2d_ref[e * CAP:(e + 1) * CAP, :] = jnp.dot(
                xg_ref[e * CAP:(e + 1) * CAP, :],
                w_ref[e],
                preferred_element_type=jnp.float32,
            ).astype(rows2d_ref.dtype)
        for row in range(ROWS):
            rows_ref[row, :, :] = rows2d_ref[pl.ds(row, 1), :]

        out_ref[:, :, :] = jnp.zeros((TOK_PER_DEV, 1, D_HID), out_ref.dtype)

        for row in range(ROWS):
            t = tok_ref[row]
            valid = t < N_TOK
            ts = jnp.where(valid, t, 0)
            dest = ts // TOK_PER_DEV
            j = lax.rem(ts, TOK_PER_DEV)

            @pl.when(valid & (dest == my))
            def _(row=row, j=j):
                out_ref[pl.ds(j, 1)] = rows_ref[pl.ds(row, 1)]

        barrier_sem = pltpu.get_barrier_semaphore()
        for k in range(1, N_DEV):
            pl.semaphore_signal(
                barrier_sem, inc=1,
                device_id=(lax.rem(my + k, N_DEV),),
                device_id_type=pl.DeviceIdType.MESH,
            )
        pl.semaphore_wait(barrier_sem, N_DEV - 1)

        for row in range(ROWS):
            t = tok_ref[row]
            valid = t < N_TOK
            ts = jnp.where(valid, t, 0)
            dest = ts // TOK_PER_DEV
            j = lax.rem(ts, TOK_PER_DEV)

            @pl.when(valid & (dest != my))
            def _(row=row, j=j, dest=dest):
                pltpu.make_async_remote_copy(
                    src_ref=rows_ref.at[pl.ds(row, 1)],
                    dst_ref=out_ref.at[pl.ds(j, 1)],
                    send_sem=send_sems.at[row],
                    recv_sem=recv_sems.at[j],
                    device_id=(dest,),
                    device_id_type=pl.DeviceIdType.MESH,
                ).start()

        for j in range(TOK_PER_DEV):
            @pl.when(exp_ref[j] != 0)
            def _(j=j):
                pltpu.make_async_remote_copy(
                    src_ref=rows_ref.at[pl.ds(0, 1)],
                    dst_ref=out_ref.at[pl.ds(j, 1)],
                    send_sem=send_sems.at[0],
                    recv_sem=recv_sems.at[j],
                    device_id=(my,),
                    device_id_type=pl.DeviceIdType.MESH,
                ).wait_recv()

        for row in range(ROWS):
            t = tok_ref[row]
            valid = t < N_TOK
            ts = jnp.where(valid, t, 0)
            dest = ts // TOK_PER_DEV

            @pl.when(valid & (dest != my))
            def _(row=row):
                pltpu.make_async_remote_copy(
                    src_ref=rows_ref.at[pl.ds(row, 1)],
                    dst_ref=out_ref.at[pl.ds(0, 1)],
                    send_sem=send_sems.at[row],
                    recv_sem=recv_sems.at[0],
                    device_id=(my,),
                    device_id_type=pl.DeviceIdType.MESH,
                ).wait_send()

        @functools.partial(
            pl.run_scoped, second_barrier=pltpu.SemaphoreType.REGULAR
        )
        def _(second_barrier):
            for k in range(1, N_DEV):
                pl.semaphore_signal(
                    second_barrier, inc=1,
                    device_id=(lax.rem(my + k, N_DEV),),
                    device_id_type=pl.DeviceIdType.MESH,
                )
            pl.semaphore_wait(second_barrier, N_DEV - 1)

    return pl.pallas_call(
        body,
        out_shape=jax.ShapeDtypeStruct((TOK_PER_DEV, 1, D_HID), jnp.bfloat16),
        in_specs=[
            pl.BlockSpec(memory_space=pltpu.SMEM),
            pl.BlockSpec(memory_space=pltpu.SMEM),
            pl.BlockSpec(memory_space=pltpu.VMEM),
            pl.BlockSpec(memory_space=pltpu.VMEM),
        ],
        out_specs=pl.BlockSpec(memory_space=pltpu.VMEM),
        scratch_shapes=[
            pltpu.VMEM((ROWS, D_HID), jnp.bfloat16),
            pltpu.VMEM((ROWS, 1, D_HID), jnp.bfloat16),
            pltpu.SemaphoreType.DMA((ROWS,)),
            pltpu.SemaphoreType.DMA((TOK_PER_DEV,)),
        ],
        compiler_params=pltpu.CompilerParams(collective_id=0),
    )(my_tok, expect, xg, w)


def kernel(x, router_W, route_idx, expert_W):
    del router_W

    my = lax.axis_index("i")
    r = route_idx[:, 0].astype(jnp.int32)

    onehot = r[:, None] == jnp.arange(N_EXP, dtype=jnp.int32)[None, :]
    cum = jnp.cumsum(onehot.astype(jnp.int32), axis=0) - 1
    rank = jnp.sum(jnp.where(onehot, cum, 0), axis=1)

    my_experts = my * EXP_PER_DEV + jnp.arange(EXP_PER_DEV, dtype=jnp.int32)
    slot_mask = (
        (r[None, None, :] == my_experts[:, None, None])
        & (rank[None, None, :] == jnp.arange(CAP, dtype=jnp.int32)[None, :, None])
    )
    my_tok = jnp.where(
        jnp.any(slot_mask, axis=-1),
        jnp.argmax(slot_mask, axis=-1).astype(jnp.int32),
        N_TOK,
    ).reshape(ROWS)

    valid = my_tok < N_TOK
    xg = x[jnp.where(valid, my_tok, 0)] * valid[:, None].astype(x.dtype)

    t0 = my * TOK_PER_DEV
    e_mine = lax.dynamic_slice(r, (t0,), (TOK_PER_DEV,))
    rank_mine = lax.dynamic_slice(rank, (t0,), (TOK_PER_DEV,))
    active = rank_mine < CAP
    remote = active & (e_mine // EXP_PER_DEV != my)
    expect = remote.astype(jnp.int32)

    out = _compute_and_scatter(my_tok, expect, xg, expert_W)
    return out.reshape(TOK_PER_DEV, D_HID).astype(jnp.float32)


# device time: 32852 ns/iter; 2.5644x vs baseline; 1.0257x over previous
import functools

import jax
import jax.numpy as jnp
from jax import lax
from jax.experimental import pallas as pl
from jax.experimental.pallas import tpu as pltpu

N_DEV = 16
N_TOK = 2048
D_MODEL = 512
D_HID = 1024
N_EXP = 128
EXP_PER_DEV = N_EXP // N_DEV
CAP = 12
ROWS = EXP_PER_DEV * CAP
TOK_PER_DEV = N_TOK // N_DEV


def _compute_and_scatter(my_tok, expect, xg, w):

    def body(tok_ref, exp_ref, xg_ref, w_hbm_ref, out_ref, w_ref,
             rows2d_ref, rows_ref, send_sems, recv_sems, w_sems):
        my = lax.axis_index("i")

        w_dmas = []
        for e in range(EXP_PER_DEV):
            dma = pltpu.make_async_copy(
                w_hbm_ref.at[e], w_ref.at[e], w_sems.at[e]
            )
            dma.start()
            w_dmas.append(dma)

        out_ref[:, :, :] = jnp.zeros((TOK_PER_DEV, 1, D_HID), out_ref.dtype)

        barrier_sem = pltpu.get_barrier_semaphore()
        for k in range(1, N_DEV):
            pl.semaphore_signal(
                barrier_sem, inc=1,
                device_id=(lax.rem(my + k, N_DEV),),
                device_id_type=pl.DeviceIdType.MESH,
            )
        pl.semaphore_wait(barrier_sem, N_DEV - 1)

        for e in range(EXP_PER_DEV):
            w_dmas[e].wait()
            rows2d_ref[e * CAP:(e + 1) * CAP, :] = jnp.dot(
                xg_ref[e * CAP:(e + 1) * CAP, :],
                w_ref[e],
                preferred_element_type=jnp.float32,
            ).astype(rows2d_ref.dtype)
            for c in range(CAP):
                row = e * CAP + c
                rows_ref[row, :, :] = rows2d_ref[pl.ds(row, 1), :]
                t = tok_ref[row]
                valid = t < N_TOK
                ts = jnp.where(valid, t, 0)
                dest = ts // TOK_PER_DEV
                j = lax.rem(ts, TOK_PER_DEV)

                @pl.when(valid & (dest == my))
                def _(row=row, j=j):
                    out_ref[pl.ds(j, 1)] = rows_ref[pl.ds(row, 1)]

                @pl.when(valid & (dest != my))
                def _(row=row, j=j, dest=dest):
                    pltpu.make_async_remote_copy(
                        src_ref=rows_ref.at[pl.ds(row, 1)],
                        dst_ref=out_ref.at[pl.ds(j, 1)],
                        send_sem=send_sems.at[row],
                        recv_sem=recv_sems.at[j],
                        device_id=(dest,),
                        device_id_type=pl.DeviceIdType.MESH,
                    ).start()

        for j in range(TOK_PER_DEV):
            @pl.when(exp_ref[j] != 0)
            def _(j=j):
                pltpu.make_async_remote_copy(
                    src_ref=rows_ref.at[pl.ds(0, 1)],
                    dst_ref=out_ref.at[pl.ds(j, 1)],
                    send_sem=send_sems.at[0],
                    recv_sem=recv_sems.at[j],
                    device_id=(my,),
                    device_id_type=pl.DeviceIdType.MESH,
                ).wait_recv()

        for row in range(ROWS):
            t = tok_ref[row]
            valid = t < N_TOK
            ts = jnp.where(valid, t, 0)
            dest = ts // TOK_PER_DEV

            @pl.when(valid & (dest != my))
            def _(row=row):
                pltpu.make_async_remote_copy(
                    src_ref=rows_ref.at[pl.ds(row, 1)],
                    dst_ref=out_ref.at[pl.ds(0, 1)],
                    send_sem=send_sems.at[row],
                    recv_sem=recv_sems.at[0],
                    device_id=(my,),
                    device_id_type=pl.DeviceIdType.MESH,
                ).wait_send()

        @functools.partial(
            pl.run_scoped, second_barrier=pltpu.SemaphoreType.REGULAR
        )
        def _(second_barrier):
            for k in range(1, N_DEV):
                pl.semaphore_signal(
                    second_barrier, inc=1,
                    device_id=(lax.rem(my + k, N_DEV),),
                    device_id_type=pl.DeviceIdType.MESH,
                )
            pl.semaphore_wait(second_barrier, N_DEV - 1)

    return pl.pallas_call(
        body,
        out_shape=jax.ShapeDtypeStruct((TOK_PER_DEV, 1, D_HID), jnp.bfloat16),
        in_specs=[
            pl.BlockSpec(memory_space=pltpu.SMEM),
            pl.BlockSpec(memory_space=pltpu.SMEM),
            pl.BlockSpec(memory_space=pltpu.VMEM),
            pl.BlockSpec(memory_space=pltpu.MemorySpace.HBM),
        ],
        out_specs=pl.BlockSpec(memory_space=pltpu.VMEM),
        scratch_shapes=[
            pltpu.VMEM((EXP_PER_DEV, D_MODEL, D_HID), jnp.float32),
            pltpu.VMEM((ROWS, D_HID), jnp.bfloat16),
            pltpu.VMEM((ROWS, 1, D_HID), jnp.bfloat16),
            pltpu.SemaphoreType.DMA((ROWS,)),
            pltpu.SemaphoreType.DMA((TOK_PER_DEV,)),
            pltpu.SemaphoreType.DMA((EXP_PER_DEV,)),
        ],
        compiler_params=pltpu.CompilerParams(collective_id=0),
    )(my_tok, expect, xg, w)


def kernel(x, router_W, route_idx, expert_W):
    del router_W

    my = lax.axis_index("i")
    r = route_idx[:, 0].astype(jnp.int32)

    onehot = r[:, None] == jnp.arange(N_EXP, dtype=jnp.int32)[None, :]
    cum = jnp.cumsum(onehot.astype(jnp.int32), axis=0) - 1
    rank = jnp.sum(jnp.where(onehot, cum, 0), axis=1)

    my_experts = my * EXP_PER_DEV + jnp.arange(EXP_PER_DEV, dtype=jnp.int32)
    slot_mask = (
        (r[None, None, :] == my_experts[:, None, None])
        & (rank[None, None, :] == jnp.arange(CAP, dtype=jnp.int32)[None, :, None])
    )
    my_tok = jnp.where(
        jnp.any(slot_mask, axis=-1),
        jnp.argmax(slot_mask, axis=-1).astype(jnp.int32),
        N_TOK,
    ).reshape(ROWS)

    valid = my_tok < N_TOK
    xg = x[jnp.where(valid, my_tok, 0)] * valid[:, None].astype(x.dtype)

    t0 = my * TOK_PER_DEV
    e_mine = lax.dynamic_slice(r, (t0,), (TOK_PER_DEV,))
    rank_mine = lax.dynamic_slice(rank, (t0,), (TOK_PER_DEV,))
    active = rank_mine < CAP
    remote = active & (e_mine // EXP_PER_DEV != my)
    expect = remote.astype(jnp.int32)

    out = _compute_and_scatter(my_tok, expect, xg, expert_W)
    return out.reshape(TOK_PER_DEV, D_HID)
